# Initial kernel scaffold; baseline (speedup 1.0000x reference)
import jax
import jax.numpy as jnp
from jax.experimental import pallas as pl


def kernel(x, edge_index_0, edge_index_1, Wl0, bl0, Wr0, Wl1, bl1, Wr1):
    raise NotImplementedError("write your pallas kernel here")


# speedup vs baseline: 8.9856x; 8.9856x over previous
import functools

import jax
import jax.numpy as jnp
from jax import lax
from jax.experimental import pallas as pl
from jax.experimental.pallas import tpu as pltpu
from jax.experimental.pallas import tpu_sc as plsc

N0, N1, N2 = 50000, 10000, 2048
E0, E1 = 320000, 65536
D = 128
NC, NS = 2, 16
NW = NC * NS
L = 16


def _matmul_nt_body(x_ref, w_ref, o_ref):
    o_ref[...] = lax.dot_general(
        x_ref[...], w_ref[...], (((1,), (1,)), ((), ())),
        preferred_element_type=jnp.float32)


def _linear_nt(x, w):
    return pl.pallas_call(
        _matmul_nt_body,
        out_shape=jax.ShapeDtypeStruct((x.shape[0], w.shape[0]), jnp.float32),
    )(x, w)


def _make_agg(E, Ntgt, K, clamp):
    EW = E // NW
    NIT = EW // K
    NtgtA = Ntgt + (L if clamp else 0)
    NtgtP = -(-NtgtA // (NS * 8)) * (NS * 8)
    RW = NtgtP // NS
    mesh = plsc.VectorSubcoreMesh(core_axis_name="c", subcore_axis_name="s")

    @functools.partial(
        pl.kernel,
        out_type=[jax.ShapeDtypeStruct((NC, NS, RW, D), jnp.float32),
                  jax.ShapeDtypeStruct((NC, NS, NtgtP), jnp.float32)],
        mesh=mesh,
        scratch_types=[
            pltpu.VMEM((NIT, K), jnp.int32),
            pltpu.VMEM((NIT, K), jnp.int32),
            pltpu.VMEM((K,), jnp.int32),
            pltpu.VMEM((K, D), jnp.float32),
            pltpu.VMEM((NtgtP,), jnp.float32),
            pltpu.VMEM_SHARED((NtgtP, D), jnp.float32),
            pltpu.SemaphoreType.DMA,
        ],
        compiler_params=pltpu.CompilerParams(needs_layout_passes=False),
    )
    def agg(z_hbm, src_hbm, dst_hbm, zeros_hbm, out_hbm, cnt_hbm,
            src_v, dst_v, dstt_v, rows_v, hist_v, acc_sh, sem):
        c = lax.axis_index("c")
        s = lax.axis_index("s")
        wid = s * NC + c
        pltpu.sync_copy(src_hbm.at[wid], src_v)
        pltpu.sync_copy(dst_hbm.at[wid], dst_v)
        pltpu.sync_copy(zeros_hbm.at[s], acc_sh.at[pl.ds(s * RW, RW)])

        def zbody(i, carry):
            hist_v[pl.ds(i * L, L)] = jnp.zeros((L,), jnp.float32)
            return carry

        lax.fori_loop(0, NtgtP // L, zbody, 0)
        plsc.subcore_barrier()

        ones = jnp.ones((L,), jnp.float32)

        def body(i, carry):
            cp = pltpu.async_copy(z_hbm.at[src_v.at[i]], rows_v, sem)
            for j in range(K // L):
                d16 = dst_v[i, pl.ds(j * L, L)]
                if clamp:
                    d16 = jnp.minimum(d16, Ntgt)
                    dstt_v[pl.ds(j * L, L)] = d16
                plsc.addupdate_scatter(hist_v, [d16], ones)
            cp.wait()
            didx = dstt_v if clamp else dst_v.at[i]
            pltpu.sync_copy(rows_v, acc_sh.at[didx], add=True)
            return carry

        lax.fori_loop(0, NIT, body, 0)
        pltpu.sync_copy(hist_v, cnt_hbm.at[c].at[s])
        plsc.subcore_barrier()
        pltpu.sync_copy(acc_sh.at[pl.ds(s * RW, RW)], out_hbm.at[c].at[s])

    return agg


def _mid_body(p_ref, c_ref, x_ref, wr_ref, bl_ref, wl1_ref, h_ref, z1_ref):
    s = p_ref[0] + p_ref[1]
    cnt = jnp.maximum(jnp.sum(c_ref[...], axis=0), 1.0)
    mean = s / cnt[:, None]
    h = mean + bl_ref[...] + lax.dot_general(
        x_ref[...], wr_ref[...], (((1,), (1,)), ((), ())),
        preferred_element_type=jnp.float32)
    h = jnp.maximum(h, 0.0)
    h_ref[...] = h
    z1_ref[...] = lax.dot_general(
        h, wl1_ref[...], (((1,), (1,)), ((), ())),
        preferred_element_type=jnp.float32)


def _final_body(p_ref, c_ref, h_ref, wr_ref, bl_ref, o_ref):
    s = p_ref[0] + p_ref[1]
    cnt = jnp.maximum(jnp.sum(c_ref[...], axis=0), 1.0)
    mean = s / cnt[:, None]
    o_ref[...] = mean + bl_ref[...] + lax.dot_general(
        h_ref[...], wr_ref[...], (((1,), (1,)), ((), ())),
        preferred_element_type=jnp.float32)


def kernel(x, edge_index_0, edge_index_1, Wl0, bl0, Wr0, Wl1, bl1, Wr1):
    x = x.astype(jnp.float32)
    e0 = edge_index_0.astype(jnp.int32)
    e1 = edge_index_1.astype(jnp.int32)

    K0, K1 = 80, 128
    src0 = e0[0].reshape(NW, E0 // NW // K0, K0)
    dst0 = e0[1].reshape(NW, E0 // NW // K0, K0)
    src1 = e1[0].reshape(NW, E1 // NW // K1, K1)
    dst1 = e1[1].reshape(NW, E1 // NW // K1, K1)

    z0 = _linear_nt(x[:N1], Wl0)
    N2P = -(-(N2 + L) // (NS * 8)) * (NS * 8)
    p0, c0 = _make_agg(E0, N2, K0, True)(
        z0, src0, dst0, jnp.zeros((NS, N2P // NS, D), jnp.float32))
    p0 = p0.reshape(NC, N2P, D)

    h, z1c = pl.pallas_call(
        _mid_body,
        out_shape=[jax.ShapeDtypeStruct((N2, D), jnp.float32),
                   jax.ShapeDtypeStruct((N2, D), jnp.float32)],
    )(p0[:, :N2, :], c0.reshape(NW, N2P)[:, :N2], x[:N2], Wr0,
      bl0.reshape(1, D), Wl1)

    p1, c1 = _make_agg(E1, N2, K1, False)(
        z1c, src1, dst1, jnp.zeros((NS, N2 // NS, D), jnp.float32))
    p1 = p1.reshape(NC, N2, D)

    out = pl.pallas_call(
        _final_body,
        out_shape=jax.ShapeDtypeStruct((N2, D), jnp.float32),
    )(p1, c1.reshape(NW, N2), h, Wr1, bl1.reshape(1, D))
    return out


# speedup vs baseline: 12.1431x; 1.3514x over previous
import functools

import jax
import jax.numpy as jnp
from jax import lax
from jax.experimental import pallas as pl
from jax.experimental.pallas import tpu as pltpu
from jax.experimental.pallas import tpu_sc as plsc

N0, N1, N2 = 50000, 10000, 2048
E0, E1 = 320000, 65536
D = 128
NC, NS = 2, 16
NW = NC * NS
L = 16


def _matmul_nt_body(x_ref, w_ref, o_ref):
    o_ref[...] = lax.dot_general(
        x_ref[...], w_ref[...], (((1,), (1,)), ((), ())),
        preferred_element_type=jnp.float32)


def _linear_nt(x, w):
    return pl.pallas_call(
        _matmul_nt_body,
        out_shape=jax.ShapeDtypeStruct((x.shape[0], w.shape[0]), jnp.float32),
    )(x, w)


def _make_agg(E, Ntgt, K, clamp):
    EW = E // NW
    NIT = EW // K
    NtgtA = Ntgt + (L if clamp else 0)
    NtgtP = -(-NtgtA // (NS * 8)) * (NS * 8)
    RW = NtgtP // NS
    mesh = plsc.VectorSubcoreMesh(core_axis_name="c", subcore_axis_name="s")

    @functools.partial(
        pl.kernel,
        out_type=[jax.ShapeDtypeStruct((NC, NS, RW, D), jnp.float32),
                  jax.ShapeDtypeStruct((NC, NS, NtgtP), jnp.float32)],
        mesh=mesh,
        scratch_types=[
            pltpu.VMEM((NIT, K), jnp.int32),
            pltpu.VMEM((NIT, K), jnp.int32),
            pltpu.VMEM((K,), jnp.int32),
            pltpu.VMEM((K, D), jnp.float32),
            pltpu.VMEM((NtgtP,), jnp.float32),
            pltpu.VMEM_SHARED((NtgtP, D), jnp.float32),
            pltpu.SemaphoreType.DMA,
        ],
        compiler_params=pltpu.CompilerParams(needs_layout_passes=False),
    )
    def agg(z_hbm, src_hbm, dst_hbm, zeros_hbm, out_hbm, cnt_hbm,
            src_v, dst_v, dstt_v, rows_v, hist_v, acc_sh, sem):
        c = lax.axis_index("c")
        s = lax.axis_index("s")
        wid = s * NC + c
        pltpu.sync_copy(src_hbm.at[wid], src_v)
        pltpu.sync_copy(dst_hbm.at[wid], dst_v)
        pltpu.sync_copy(zeros_hbm.at[s], acc_sh.at[pl.ds(s * RW, RW)])

        def zbody(i, carry):
            hist_v[pl.ds(i * L, L)] = jnp.zeros((L,), jnp.float32)
            return carry

        lax.fori_loop(0, NtgtP // L, zbody, 0)
        plsc.subcore_barrier()

        ones = jnp.ones((L,), jnp.float32)

        def body(i, carry):
            cp = pltpu.async_copy(z_hbm.at[src_v.at[i]], rows_v, sem)
            for j in range(K // L):
                d16 = dst_v[i, pl.ds(j * L, L)]
                if clamp:
                    d16 = jnp.minimum(d16, Ntgt)
                    dstt_v[pl.ds(j * L, L)] = d16
                plsc.addupdate_scatter(hist_v, [d16], ones)
            cp.wait()
            didx = dstt_v if clamp else dst_v.at[i]
            pltpu.sync_copy(rows_v, acc_sh.at[didx], add=True)
            return carry

        lax.fori_loop(0, NIT, body, 0)
        pltpu.sync_copy(hist_v, cnt_hbm.at[c].at[s])
        plsc.subcore_barrier()
        pltpu.sync_copy(acc_sh.at[pl.ds(s * RW, RW)], out_hbm.at[c].at[s])

    return agg


def _make_agg_filter(E, Ntgt, K):
    EW = E // NW
    NG = EW // L
    CB = (EW + 2 * K - 1) // K
    NtgtP = -(-(Ntgt + L) // (NS * 8)) * (NS * 8)
    RW = NtgtP // NS
    KSH = K.bit_length() - 1
    assert K == 1 << KSH
    mesh = plsc.VectorSubcoreMesh(core_axis_name="c", subcore_axis_name="s")

    @functools.partial(
        pl.kernel,
        out_type=[jax.ShapeDtypeStruct((NC, NS, RW, D), jnp.float32),
                  jax.ShapeDtypeStruct((NC, NS, Ntgt), jnp.float32)],
        mesh=mesh,
        scratch_types=[
            pltpu.VMEM((EW,), jnp.int32),
            pltpu.VMEM((EW,), jnp.int32),
            pltpu.VMEM((CB, K), jnp.int32),
            pltpu.VMEM((CB, K), jnp.int32),
            pltpu.VMEM((K, D), jnp.float32),
            pltpu.VMEM((Ntgt,), jnp.float32),
            pltpu.VMEM_SHARED((NtgtP, D), jnp.float32),
            pltpu.SemaphoreType.DMA,
        ],
        compiler_params=pltpu.CompilerParams(needs_layout_passes=False),
    )
    def agg(z_hbm, src_hbm, dst_hbm, zeros_hbm, out_hbm, cnt_hbm,
            src_v, dst_v, srcc_v, dstc_v, rows_v, hist_v, acc_sh, sem):
        c = lax.axis_index("c")
        s = lax.axis_index("s")
        wid = s * NC + c
        pltpu.sync_copy(src_hbm.at[wid], src_v)
        pltpu.sync_copy(dst_hbm.at[wid], dst_v)
        pltpu.sync_copy(zeros_hbm.at[s], acc_sh.at[pl.ds(s * RW, RW)])

        def zbody(i, carry):
            hist_v[pl.ds(i * L, L)] = jnp.zeros((L,), jnp.float32)
            return carry

        lax.fori_loop(0, Ntgt // L, zbody, 0)

        ones_f = jnp.ones((L,), jnp.float32)
        lanes = lax.iota(jnp.int32, L)

        def fbody(g, off):
            d16 = dst_v[pl.ds(g * L, L)]
            s16 = src_v[pl.ds(g * L, L)]
            m = d16 < Ntgt
            mi = m.astype(jnp.int32)
            pos = off + plsc.cumsum(mi) - 1
            plsc.store_scatter(srcc_v, [pos >> KSH, pos & (K - 1)], s16, mask=m)
            plsc.store_scatter(dstc_v, [pos >> KSH, pos & (K - 1)], d16, mask=m)
            plsc.addupdate_scatter(hist_v, [d16], ones_f, mask=m)
            return off + plsc.all_reduce_population_count(m)

        off = lax.fori_loop(0, NG, fbody, jnp.zeros((L,), jnp.int32))
        nkeep = jnp.max(off)

        for j in range(K // L):
            pos = nkeep + j * L + lanes
            plsc.store_scatter(srcc_v, [pos >> KSH, pos & (K - 1)],
                               jnp.zeros((L,), jnp.int32))
            plsc.store_scatter(dstc_v, [pos >> KSH, pos & (K - 1)],
                               jnp.full((L,), Ntgt, jnp.int32))
        plsc.subcore_barrier()

        trips = (nkeep + K - 1) >> KSH

        def body(t, carry):
            pltpu.async_copy(z_hbm.at[srcc_v.at[t]], rows_v, sem).wait()
            pltpu.sync_copy(rows_v, acc_sh.at[dstc_v.at[t]], add=True)
            return carry

        lax.fori_loop(0, trips, body, 0)
        pltpu.sync_copy(hist_v, cnt_hbm.at[c].at[s])
        plsc.subcore_barrier()
        pltpu.sync_copy(acc_sh.at[pl.ds(s * RW, RW)], out_hbm.at[c].at[s])

    return agg


def _mid_body(p_ref, c_ref, x_ref, wr_ref, bl_ref, wl1_ref, h_ref, z1_ref):
    s = p_ref[0] + p_ref[1]
    cnt = jnp.maximum(jnp.sum(c_ref[...], axis=0), 1.0)
    mean = s / cnt[:, None]
    h = mean + bl_ref[...] + lax.dot_general(
        x_ref[...], wr_ref[...], (((1,), (1,)), ((), ())),
        preferred_element_type=jnp.float32)
    h = jnp.maximum(h, 0.0)
    h_ref[...] = h
    z1_ref[...] = lax.dot_general(
        h, wl1_ref[...], (((1,), (1,)), ((), ())),
        preferred_element_type=jnp.float32)


def _final_body(p_ref, c_ref, h_ref, wr_ref, bl_ref, o_ref):
    s = p_ref[0] + p_ref[1]
    cnt = jnp.maximum(jnp.sum(c_ref[...], axis=0), 1.0)
    mean = s / cnt[:, None]
    o_ref[...] = mean + bl_ref[...] + lax.dot_general(
        h_ref[...], wr_ref[...], (((1,), (1,)), ((), ())),
        preferred_element_type=jnp.float32)


def kernel(x, edge_index_0, edge_index_1, Wl0, bl0, Wr0, Wl1, bl1, Wr1):
    x = x.astype(jnp.float32)
    e0 = edge_index_0.astype(jnp.int32)
    e1 = edge_index_1.astype(jnp.int32)

    K0, K1 = 128, 128
    src0 = e0[0].reshape(NW, E0 // NW)
    dst0 = e0[1].reshape(NW, E0 // NW)
    src1 = e1[0].reshape(NW, E1 // NW // K1, K1)
    dst1 = e1[1].reshape(NW, E1 // NW // K1, K1)

    z0 = _linear_nt(x[:N1], Wl0)
    N2P = -(-(N2 + L) // (NS * 8)) * (NS * 8)
    p0, c0 = _make_agg_filter(E0, N2, K0)(
        z0, src0, dst0, jnp.zeros((NS, N2P // NS, D), jnp.float32))
    p0 = p0.reshape(NC, N2P, D)

    h, z1c = pl.pallas_call(
        _mid_body,
        out_shape=[jax.ShapeDtypeStruct((N2, D), jnp.float32),
                   jax.ShapeDtypeStruct((N2, D), jnp.float32)],
    )(p0[:, :N2, :], c0.reshape(NW, N2), x[:N2], Wr0,
      bl0.reshape(1, D), Wl1)

    p1, c1 = _make_agg(E1, N2, K1, False)(
        z1c, src1, dst1, jnp.zeros((NS, N2 // NS, D), jnp.float32))
    p1 = p1.reshape(NC, N2, D)

    out = pl.pallas_call(
        _final_body,
        out_shape=jax.ShapeDtypeStruct((N2, D), jnp.float32),
    )(p1, c1.reshape(NW, N2), h, Wr1, bl1.reshape(1, D))
    return out


# speedup vs baseline: 15.0341x; 1.2381x over previous
import functools

import jax
import jax.numpy as jnp
from jax import lax
from jax.experimental import pallas as pl
from jax.experimental.pallas import tpu as pltpu
from jax.experimental.pallas import tpu_sc as plsc

N0, N1, N2 = 50000, 10000, 2048
E0, E1 = 320000, 65536
D = 128
NC, NS = 2, 16
NW = NC * NS
L = 16
K = 128
FU = 4


def _make_agg(E, Ntgt):
    EW = E // NW
    NG = EW // L
    CB = (EW + 2 * K - 1) // K
    NtgtP = -(-(Ntgt + L) // (NS * 8)) * (NS * 8)
    RW = NtgtP // NS
    KSH = K.bit_length() - 1
    mesh = plsc.VectorSubcoreMesh(core_axis_name="c", subcore_axis_name="s")

    @functools.partial(
        pl.kernel,
        out_type=[jax.ShapeDtypeStruct((NC, NS, RW, D), jnp.float32),
                  jax.ShapeDtypeStruct((NC, NS, Ntgt), jnp.float32)],
        mesh=mesh,
        scratch_types=[
            pltpu.VMEM((EW,), jnp.int32),
            pltpu.VMEM((EW,), jnp.int32),
            pltpu.VMEM((CB, K), jnp.int32),
            pltpu.VMEM((CB, K), jnp.int32),
            pltpu.VMEM((K, D), jnp.float32),
            pltpu.VMEM((K, D), jnp.float32),
            pltpu.VMEM((Ntgt,), jnp.float32),
            pltpu.VMEM_SHARED((NtgtP, D), jnp.float32),
            pltpu.SemaphoreType.DMA,
            pltpu.SemaphoreType.DMA,
        ],
        compiler_params=pltpu.CompilerParams(needs_layout_passes=False),
    )
    def agg(z_hbm, se_hbm, zeros_hbm, out_hbm, cnt_hbm,
            src_v, dst_v, srcc_v, dstc_v, rows0_v, rows1_v, hist_v,
            acc_sh, sem0, sem1):
        c = lax.axis_index("c")
        s = lax.axis_index("s")
        wid = s * NC + c
        pltpu.sync_copy(se_hbm.at[0].at[wid], src_v)
        pltpu.sync_copy(se_hbm.at[1].at[wid], dst_v)
        pltpu.sync_copy(zeros_hbm.at[s], acc_sh.at[pl.ds(s * RW, RW)])

        def zbody(i, carry):
            hist_v[pl.ds(i * L, L)] = jnp.zeros((L,), jnp.float32)
            return carry

        lax.fori_loop(0, Ntgt // L, zbody, 0)

        ones_f = jnp.ones((L,), jnp.float32)
        lanes = lax.iota(jnp.int32, L)

        def fgroup(g, off):
            d16 = dst_v[pl.ds(g * L, L)]
            s16 = src_v[pl.ds(g * L, L)]
            m = d16 < Ntgt
            pos = off + plsc.cumsum(m.astype(jnp.int32)) - 1
            plsc.store_scatter(
                srcc_v, [pos >> KSH, pos & (K - 1)], s16, mask=m)
            plsc.store_scatter(
                dstc_v, [pos >> KSH, pos & (K - 1)], d16, mask=m)
            plsc.addupdate_scatter(hist_v, [d16], ones_f, mask=m)
            return off + plsc.all_reduce_population_count(m)

        def fbody(g0, off):
            for u in range(FU):
                off = fgroup(g0 * FU + u, off)
            return off

        off = lax.fori_loop(0, NG // FU, fbody, jnp.zeros((L,), jnp.int32))
        for g in range(NG - NG % FU, NG):
            off = fgroup(g, off)
        nkeep = jnp.max(off)

        for j in range(K // L):
            pos = nkeep + j * L + lanes
            plsc.store_scatter(srcc_v, [pos >> KSH, pos & (K - 1)],
                               jnp.zeros((L,), jnp.int32))
            plsc.store_scatter(dstc_v, [pos >> KSH, pos & (K - 1)],
                               jnp.full((L,), Ntgt, jnp.int32))
        pltpu.sync_copy(hist_v, cnt_hbm.at[c].at[s])
        plsc.subcore_barrier()

        trips = jnp.maximum((nkeep + K - 1) >> KSH, 1)

        pltpu.async_copy(z_hbm.at[srcc_v.at[0]], rows0_v, sem0)

        def body(tt, carry):
            for b in range(2):
                t = 2 * tt + b
                buf, sem = (rows0_v, sem0) if b == 0 else (rows1_v, sem1)
                obuf, osem = (rows1_v, sem1) if b == 0 else (rows0_v, sem0)

                @pl.when(t < trips)
                def _process():
                    @pl.when(t + 1 < trips)
                    def _prefetch():
                        pltpu.async_copy(
                            z_hbm.at[srcc_v.at[t + 1]], obuf, osem)
                    pltpu.make_async_copy(
                        z_hbm.at[srcc_v.at[t]], buf, sem).wait()
                    pltpu.sync_copy(buf, acc_sh.at[dstc_v.at[t]], add=True)
            return carry

        lax.fori_loop(0, (trips + 1) >> 1, body, 0)
        plsc.subcore_barrier()
        pltpu.sync_copy(acc_sh.at[pl.ds(s * RW, RW)], out_hbm.at[c].at[s])

    return agg


def _full(shape):
    nd = len(shape)
    return pl.BlockSpec(shape, lambda i: (0,) * nd)


def _matmul_nt_body(x_ref, w_ref, o_ref):
    o_ref[...] = lax.dot_general(
        x_ref[...], w_ref[...], (((1,), (1,)), ((), ())),
        preferred_element_type=jnp.float32)


def _mid_body(p_ref, c_ref, x_ref, wr_ref, bl_ref, wl1_ref, h_ref, z1_ref):
    sm = p_ref[0] + p_ref[1]
    cnt = jnp.maximum(jnp.sum(c_ref[...], axis=0), 1.0)
    mean = sm / cnt[:, None]
    h = mean + bl_ref[...] + lax.dot_general(
        x_ref[...], wr_ref[...], (((1,), (1,)), ((), ())),
        preferred_element_type=jnp.float32)
    h = jnp.maximum(h, 0.0)
    h_ref[...] = h
    z1_ref[...] = lax.dot_general(
        h, wl1_ref[...], (((1,), (1,)), ((), ())),
        preferred_element_type=jnp.float32)


def _final_body(p_ref, c_ref, h_ref, wr_ref, bl_ref, o_ref):
    sm = p_ref[0] + p_ref[1]
    cnt = jnp.maximum(jnp.sum(c_ref[...], axis=0), 1.0)
    mean = sm / cnt[:, None]
    o_ref[...] = mean + bl_ref[...] + lax.dot_general(
        h_ref[...], wr_ref[...], (((1,), (1,)), ((), ())),
        preferred_element_type=jnp.float32)


def kernel(x, edge_index_0, edge_index_1, Wl0, bl0, Wr0, Wl1, bl1, Wr1):
    x = x.astype(jnp.float32)
    e0 = edge_index_0.astype(jnp.int32).reshape(2, NW, E0 // NW)
    e1 = edge_index_1.astype(jnp.int32).reshape(2, NW, E1 // NW)

    N2P = -(-(N2 + L) // (NS * 8)) * (NS * 8)
    zeros = jnp.zeros((NS, N2P // NS, D), jnp.float32)

    z0 = pl.pallas_call(
        _matmul_nt_body,
        grid=(1,),
        in_specs=[_full((N1, D)), _full((D, D))],
        out_specs=_full((N1, D)),
        out_shape=jax.ShapeDtypeStruct((N1, D), jnp.float32),
    )(x, Wl0)
    p0, c0 = _make_agg(E0, N2)(z0, e0, zeros)

    h, z1c = pl.pallas_call(
        _mid_body,
        grid=(1,),
        in_specs=[_full((NC, N2, D)), _full((NW, N2)), _full((N2, D)),
                  _full((D, D)), _full((1, D)), _full((D, D))],
        out_specs=[_full((N2, D)), _full((N2, D))],
        out_shape=[jax.ShapeDtypeStruct((N2, D), jnp.float32),
                   jax.ShapeDtypeStruct((N2, D), jnp.float32)],
    )(p0.reshape(NC, N2P, D), c0.reshape(NW, N2), x, Wr0,
      bl0.reshape(1, D), Wl1)

    p1, c1 = _make_agg(E1, N2)(z1c, e1, zeros)

    out = pl.pallas_call(
        _final_body,
        grid=(1,),
        in_specs=[_full((NC, N2, D)), _full((NW, N2)), _full((N2, D)),
                  _full((D, D)), _full((1, D))],
        out_specs=_full((N2, D)),
        out_shape=jax.ShapeDtypeStruct((N2, D), jnp.float32),
    )(p1.reshape(NC, N2P, D), c1.reshape(NW, N2), h, Wr1, bl1.reshape(1, D))
    return out


# speedup vs baseline: 15.6411x; 1.0404x over previous
import functools

import jax
import jax.numpy as jnp
from jax import lax
from jax.experimental import pallas as pl
from jax.experimental.pallas import tpu as pltpu
from jax.experimental.pallas import tpu_sc as plsc

N0, N1, N2 = 50000, 10000, 2048
E0, E1 = 320000, 65536
D = 128
NC, NS = 2, 16
NW = NC * NS
L = 16
K = 128
FU = 4


def _make_agg(E, Ntgt, filt):
    EW = E // NW
    NG = EW // L
    NIT = EW // K
    CB = (EW + 2 * K - 1) // K
    NtgtP = -(-(Ntgt + L) // (NS * 8)) * (NS * 8)
    RW = NtgtP // NS
    KSH = K.bit_length() - 1
    mesh = plsc.VectorSubcoreMesh(core_axis_name="c", subcore_axis_name="s")

    idx_scratch = (
        [pltpu.VMEM((EW,), jnp.int32),
         pltpu.VMEM((EW,), jnp.int32),
         pltpu.VMEM((CB, K), jnp.int32),
         pltpu.VMEM((CB, K), jnp.int32)]
        if filt else
        [pltpu.VMEM((NIT, K), jnp.int32),
         pltpu.VMEM((NIT, K), jnp.int32)])

    @functools.partial(
        pl.kernel,
        out_type=[jax.ShapeDtypeStruct((NC, NS, RW, D), jnp.float32),
                  jax.ShapeDtypeStruct((NC, NS, NtgtP), jnp.float32)],
        mesh=mesh,
        scratch_types=idx_scratch + [
            pltpu.VMEM((K, D), jnp.float32),
            pltpu.VMEM((K, D), jnp.float32),
            pltpu.VMEM((NtgtP,), jnp.float32),
            pltpu.VMEM_SHARED((NtgtP, D), jnp.float32),
            pltpu.SemaphoreType.DMA,
            pltpu.SemaphoreType.DMA,
        ],
        compiler_params=pltpu.CompilerParams(needs_layout_passes=False),
    )
    def agg(z_hbm, se_hbm, zeros_hbm, zeros1_hbm, out_hbm, cnt_hbm, *refs):
        if filt:
            (src_v, dst_v, srcc_v, dstc_v, rows0_v, rows1_v, hist_v,
             acc_sh, sem0, sem1) = refs
        else:
            (srcc_v, dstc_v, rows0_v, rows1_v, hist_v,
             acc_sh, sem0, sem1) = refs
        c = lax.axis_index("c")
        s = lax.axis_index("s")
        wid = s * NC + c
        if filt:
            pltpu.sync_copy(se_hbm.at[0].at[wid], src_v)
            pltpu.sync_copy(se_hbm.at[1].at[wid], dst_v)
        else:
            pltpu.sync_copy(se_hbm.at[0].at[wid], srcc_v)
            pltpu.sync_copy(se_hbm.at[1].at[wid], dstc_v)
        pltpu.sync_copy(zeros_hbm.at[s], acc_sh.at[pl.ds(s * RW, RW)])
        pltpu.sync_copy(zeros1_hbm, hist_v)

        ones_f = jnp.ones((L,), jnp.float32)

        if filt:
            def fgroup(g, off):
                d16 = dst_v[pl.ds(g * L, L)]
                s16 = src_v[pl.ds(g * L, L)]
                m = d16 < Ntgt
                pos = off + plsc.cumsum(m.astype(jnp.int32)) - 1
                plsc.store_scatter(
                    srcc_v, [pos >> KSH, pos & (K - 1)], s16, mask=m)
                plsc.store_scatter(
                    dstc_v, [pos >> KSH, pos & (K - 1)], d16, mask=m)
                return off + plsc.all_reduce_population_count(m)

            off = plsc.parallel_loop(
                0, NG, unroll=FU, carry=jnp.zeros((L,), jnp.int32))(fgroup)
            nkeep = jnp.max(off)

            lanes = lax.iota(jnp.int32, L)
            for j in range(K // L):
                pos = nkeep + j * L + lanes
                plsc.store_scatter(srcc_v, [pos >> KSH, pos & (K - 1)],
                                   jnp.zeros((L,), jnp.int32))
                plsc.store_scatter(dstc_v, [pos >> KSH, pos & (K - 1)],
                                   jnp.full((L,), Ntgt, jnp.int32))
            trips = jnp.maximum((nkeep + K - 1) >> KSH, 1)
        else:
            trips = NIT
        plsc.subcore_barrier()

        pltpu.async_copy(z_hbm.at[srcc_v.at[0]], rows0_v, sem0)

        def body(tt, carry):
            for b in range(2):
                t = 2 * tt + b
                buf, sem = (rows0_v, sem0) if b == 0 else (rows1_v, sem1)
                obuf, osem = (rows1_v, sem1) if b == 0 else (rows0_v, sem0)

                @pl.when(t < trips)
                def _process():
                    @pl.when(t + 1 < trips)
                    def _prefetch():
                        pltpu.async_copy(
                            z_hbm.at[srcc_v.at[t + 1]], obuf, osem)
                    for j in range(K // L):
                        plsc.addupdate_scatter(
                            hist_v, [dstc_v[t, pl.ds(j * L, L)]], ones_f)
                    pltpu.make_async_copy(
                        z_hbm.at[srcc_v.at[t]], buf, sem).wait()
                    pltpu.sync_copy(buf, acc_sh.at[dstc_v.at[t]], add=True)
            return carry

        lax.fori_loop(0, (trips + 1) >> 1, body, 0)
        pltpu.sync_copy(hist_v, cnt_hbm.at[c].at[s])
        plsc.subcore_barrier()
        pltpu.sync_copy(acc_sh.at[pl.ds(s * RW, RW)], out_hbm.at[c].at[s])

    return agg


def _full(shape):
    nd = len(shape)
    return pl.BlockSpec(shape, lambda i: (0,) * nd)


def _matmul_nt_body(x_ref, w_ref, o_ref):
    o_ref[...] = lax.dot_general(
        x_ref[...], w_ref[...], (((1,), (1,)), ((), ())),
        preferred_element_type=jnp.float32)


def _mid_body(p_ref, c_ref, x_ref, wr_ref, bl_ref, wl1_ref, h_ref, z1_ref):
    sm = p_ref[0] + p_ref[1]
    cnt = jnp.maximum(jnp.sum(c_ref[...], axis=0), 1.0)
    mean = sm / cnt[:, None]
    h = mean + bl_ref[...] + lax.dot_general(
        x_ref[...], wr_ref[...], (((1,), (1,)), ((), ())),
        preferred_element_type=jnp.float32)
    h = jnp.maximum(h, 0.0)
    h_ref[...] = h
    z1_ref[...] = lax.dot_general(
        h, wl1_ref[...], (((1,), (1,)), ((), ())),
        preferred_element_type=jnp.float32)


def _final_body(p_ref, c_ref, h_ref, wr_ref, bl_ref, o_ref):
    sm = p_ref[0] + p_ref[1]
    cnt = jnp.maximum(jnp.sum(c_ref[...], axis=0), 1.0)
    mean = sm / cnt[:, None]
    o_ref[...] = mean + bl_ref[...] + lax.dot_general(
        h_ref[...], wr_ref[...], (((1,), (1,)), ((), ())),
        preferred_element_type=jnp.float32)


def kernel(x, edge_index_0, edge_index_1, Wl0, bl0, Wr0, Wl1, bl1, Wr1):
    x = x.astype(jnp.float32)
    e0 = edge_index_0.astype(jnp.int32).reshape(2, NW, E0 // NW)
    e1 = edge_index_1.astype(jnp.int32).reshape(2, NW, E1 // NW // K, K)

    N2P = -(-(N2 + L) // (NS * 8)) * (NS * 8)
    zeros = jnp.zeros((NS, N2P // NS, D), jnp.float32)
    zeros1 = jnp.zeros((N2P,), jnp.float32)

    z0 = pl.pallas_call(
        _matmul_nt_body,
        grid=(1,),
        in_specs=[_full((N1, D)), _full((D, D))],
        out_specs=_full((N1, D)),
        out_shape=jax.ShapeDtypeStruct((N1, D), jnp.float32),
    )(x, Wl0)
    p0, c0 = _make_agg(E0, N2, True)(z0, e0, zeros, zeros1)

    h, z1c = pl.pallas_call(
        _mid_body,
        grid=(1,),
        in_specs=[_full((NC, N2, D)), _full((NW, N2)), _full((N2, D)),
                  _full((D, D)), _full((1, D)), _full((D, D))],
        out_specs=[_full((N2, D)), _full((N2, D))],
        out_shape=[jax.ShapeDtypeStruct((N2, D), jnp.float32),
                   jax.ShapeDtypeStruct((N2, D), jnp.float32)],
    )(p0.reshape(NC, N2P, D), c0.reshape(NW, N2P), x, Wr0,
      bl0.reshape(1, D), Wl1)

    p1, c1 = _make_agg(E1, N2, False)(z1c, e1, zeros, zeros1)

    out = pl.pallas_call(
        _final_body,
        grid=(1,),
        in_specs=[_full((NC, N2, D)), _full((NW, N2)), _full((N2, D)),
                  _full((D, D)), _full((1, D))],
        out_specs=_full((N2, D)),
        out_shape=jax.ShapeDtypeStruct((N2, D), jnp.float32),
    )(p1.reshape(NC, N2P, D), c1.reshape(NW, N2P), h, Wr1, bl1.reshape(1, D))
    return out


# speedup vs baseline: 15.9248x; 1.0181x over previous
import functools

import jax
import jax.numpy as jnp
from jax import lax
from jax.experimental import pallas as pl
from jax.experimental.pallas import tpu as pltpu
from jax.experimental.pallas import tpu_sc as plsc

N0, N1, N2 = 50000, 10000, 2048
E0, E1 = 320000, 65536
D = 128
NC, NS = 2, 16
NW = NC * NS
L = 16
K = 128
FU = 4
NB = 4


def _make_agg(E, Ntgt, filt):
    EW = E // NW
    NG = EW // L
    NIT = EW // K
    CB = (EW + 2 * K - 1) // K
    NtgtP = -(-(Ntgt + L) // (NS * 8)) * (NS * 8)
    RW = NtgtP // NS
    KSH = K.bit_length() - 1
    mesh = plsc.VectorSubcoreMesh(core_axis_name="c", subcore_axis_name="s")

    idx_scratch = (
        [pltpu.VMEM((EW,), jnp.int32),
         pltpu.VMEM((EW,), jnp.int32),
         pltpu.VMEM((CB, K), jnp.int32),
         pltpu.VMEM((CB, K), jnp.int32)]
        if filt else
        [pltpu.VMEM((NIT, K), jnp.int32),
         pltpu.VMEM((NIT, K), jnp.int32)])

    @functools.partial(
        pl.kernel,
        out_type=[jax.ShapeDtypeStruct((NC, NS, RW, D), jnp.float32),
                  jax.ShapeDtypeStruct((NC, NS, NtgtP), jnp.float32)],
        mesh=mesh,
        scratch_types=idx_scratch + [
            pltpu.VMEM((NB, K, D), jnp.float32),
            pltpu.VMEM((NtgtP,), jnp.float32),
            pltpu.VMEM_SHARED((NtgtP, D), jnp.float32),
        ] + [pltpu.SemaphoreType.DMA] * NB,
        compiler_params=pltpu.CompilerParams(needs_layout_passes=False),
    )
    def agg(z_hbm, se_hbm, zeros_hbm, zeros1_hbm, out_hbm, cnt_hbm, *refs):
        if filt:
            (src_v, dst_v, srcc_v, dstc_v, rows_v, hist_v, acc_sh,
             *sems) = refs
        else:
            (srcc_v, dstc_v, rows_v, hist_v, acc_sh, *sems) = refs
        c = lax.axis_index("c")
        s = lax.axis_index("s")
        wid = s * NC + c
        if filt:
            pltpu.sync_copy(se_hbm.at[0].at[wid], src_v)
            pltpu.sync_copy(se_hbm.at[1].at[wid], dst_v)
        else:
            pltpu.sync_copy(se_hbm.at[0].at[wid], srcc_v)
            pltpu.sync_copy(se_hbm.at[1].at[wid], dstc_v)
        pltpu.sync_copy(zeros_hbm.at[s], acc_sh.at[pl.ds(s * RW, RW)])
        pltpu.sync_copy(zeros1_hbm, hist_v)

        ones_f = jnp.ones((L,), jnp.float32)

        if filt:
            def fgroup(g, off):
                d16 = dst_v[pl.ds(g * L, L)]
                s16 = src_v[pl.ds(g * L, L)]
                m = d16 < Ntgt
                pos = off + plsc.cumsum(m.astype(jnp.int32)) - 1
                plsc.store_scatter(
                    srcc_v, [pos >> KSH, pos & (K - 1)], s16, mask=m)
                plsc.store_scatter(
                    dstc_v, [pos >> KSH, pos & (K - 1)], d16, mask=m)
                return off + plsc.all_reduce_population_count(m)

            off = plsc.parallel_loop(
                0, NG, unroll=FU, carry=jnp.zeros((L,), jnp.int32))(fgroup)
            nkeep = jnp.max(off)

            lanes = lax.iota(jnp.int32, L)
            for j in range(K // L):
                pos = nkeep + j * L + lanes
                plsc.store_scatter(srcc_v, [pos >> KSH, pos & (K - 1)],
                                   jnp.zeros((L,), jnp.int32))
                plsc.store_scatter(dstc_v, [pos >> KSH, pos & (K - 1)],
                                   jnp.full((L,), Ntgt, jnp.int32))
            trips = jnp.maximum((nkeep + K - 1) >> KSH, 1)
        else:
            trips = NIT
        plsc.subcore_barrier()

        def issue(t, b):
            pltpu.async_copy(z_hbm.at[srcc_v.at[t]], rows_v.at[b], sems[b])

        def process(t, b):
            @pl.when(t + NB - 1 < trips)
            def _prefetch():
                issue(t + NB - 1, (b + NB - 1) % NB)
            for j in range(K // L):
                plsc.addupdate_scatter(
                    hist_v, [dstc_v[t, pl.ds(j * L, L)]], ones_f)
            pltpu.make_async_copy(
                z_hbm.at[srcc_v.at[t]], rows_v.at[b], sems[b]).wait()
            pltpu.sync_copy(rows_v.at[b], acc_sh.at[dstc_v.at[t]], add=True)

        issue(0, 0)
        for b in range(1, NB - 1):
            @pl.when(b < trips)
            def _prime(b=b):
                issue(b, b)

        def body(tt, carry):
            for b in range(NB):
                t = NB * tt + b

                @pl.when(t < trips)
                def _process(t=t, b=b):
                    process(t, b)
            return carry

        lax.fori_loop(0, (trips + NB - 1) // NB, body, 0)
        pltpu.sync_copy(hist_v, cnt_hbm.at[c].at[s])
        plsc.subcore_barrier()
        pltpu.sync_copy(acc_sh.at[pl.ds(s * RW, RW)], out_hbm.at[c].at[s])

    return agg


def _full(shape):
    nd = len(shape)
    return pl.BlockSpec(shape, lambda i: (0,) * nd)


def _matmul_nt_body(x_ref, w_ref, o_ref):
    o_ref[...] = lax.dot_general(
        x_ref[...], w_ref[...], (((1,), (1,)), ((), ())),
        preferred_element_type=jnp.float32)


def _mid_body(p_ref, c_ref, x_ref, wr_ref, bl_ref, wl1_ref, h_ref, z1_ref):
    sm = p_ref[0] + p_ref[1]
    cnt = jnp.maximum(jnp.sum(c_ref[...], axis=0), 1.0)
    mean = sm / cnt[:, None]
    h = mean + bl_ref[...] + lax.dot_general(
        x_ref[...], wr_ref[...], (((1,), (1,)), ((), ())),
        preferred_element_type=jnp.float32)
    h = jnp.maximum(h, 0.0)
    h_ref[...] = h
    z1_ref[...] = lax.dot_general(
        h, wl1_ref[...], (((1,), (1,)), ((), ())),
        preferred_element_type=jnp.float32)


def _final_body(p_ref, c_ref, h_ref, wr_ref, bl_ref, o_ref):
    sm = p_ref[0] + p_ref[1]
    cnt = jnp.maximum(jnp.sum(c_ref[...], axis=0), 1.0)
    mean = sm / cnt[:, None]
    o_ref[...] = mean + bl_ref[...] + lax.dot_general(
        h_ref[...], wr_ref[...], (((1,), (1,)), ((), ())),
        preferred_element_type=jnp.float32)


def kernel(x, edge_index_0, edge_index_1, Wl0, bl0, Wr0, Wl1, bl1, Wr1):
    x = x.astype(jnp.float32)
    e0 = edge_index_0.astype(jnp.int32).reshape(2, NW, E0 // NW)
    e1 = edge_index_1.astype(jnp.int32).reshape(2, NW, E1 // NW // K, K)

    N2P = -(-(N2 + L) // (NS * 8)) * (NS * 8)
    zeros = jnp.zeros((NS, N2P // NS, D), jnp.float32)
    zeros1 = jnp.zeros((N2P,), jnp.float32)

    z0 = pl.pallas_call(
        _matmul_nt_body,
        grid=(1,),
        in_specs=[_full((N1, D)), _full((D, D))],
        out_specs=_full((N1, D)),
        out_shape=jax.ShapeDtypeStruct((N1, D), jnp.float32),
    )(x, Wl0)
    p0, c0 = _make_agg(E0, N2, True)(z0, e0, zeros, zeros1)

    h, z1c = pl.pallas_call(
        _mid_body,
        grid=(1,),
        in_specs=[_full((NC, N2, D)), _full((NW, N2)), _full((N2, D)),
                  _full((D, D)), _full((1, D)), _full((D, D))],
        out_specs=[_full((N2, D)), _full((N2, D))],
        out_shape=[jax.ShapeDtypeStruct((N2, D), jnp.float32),
                   jax.ShapeDtypeStruct((N2, D), jnp.float32)],
    )(p0.reshape(NC, N2P, D), c0.reshape(NW, N2P), x, Wr0,
      bl0.reshape(1, D), Wl1)

    p1, c1 = _make_agg(E1, N2, False)(z1c, e1, zeros, zeros1)

    out = pl.pallas_call(
        _final_body,
        grid=(1,),
        in_specs=[_full((NC, N2, D)), _full((NW, N2)), _full((N2, D)),
                  _full((D, D)), _full((1, D))],
        out_specs=_full((N2, D)),
        out_shape=jax.ShapeDtypeStruct((N2, D), jnp.float32),
    )(p1.reshape(NC, N2P, D), c1.reshape(NW, N2P), h, Wr1, bl1.reshape(1, D))
    return out
